# baseline (speedup 1.0000x reference)
import jax
import jax.numpy as jnp
from jax.experimental import pallas as pl


RBLK = 128


def _mp_kernel(seq_ref, adj_ref, w_ref, ws_ref, b_ref, out_ref):
    r_i = pl.program_id(1)
    x_blk = seq_ref[0, pl.ds(r_i * RBLK, RBLK), :]
    a_blk = (adj_ref[0] != 0).astype(jnp.float32)
    h_blk = jnp.dot(x_blk, w_ref[...], preferred_element_type=jnp.float32)
    partial = jax.lax.dot_general(
        a_blk, h_blk, (((0,), (0,)), ((), ())), preferred_element_type=jnp.float32
    )

    @pl.when(r_i == 0)
    def _init():
        self_term = jnp.dot(
            seq_ref[0], ws_ref[...], preferred_element_type=jnp.float32
        )
        out_ref[0] = partial + self_term + b_ref[...]

    @pl.when(r_i != 0)
    def _acc():
        out_ref[0] += partial


def kernel(seq, mask, adj_matrix, W, W_self, b):
    B, L, d = seq.shape
    del mask
    b2d = b.reshape(1, d)
    out = pl.pallas_call(
        _mp_kernel,
        grid=(B, L // RBLK),
        in_specs=[
            pl.BlockSpec((1, L, d), lambda i, j: (i, 0, 0)),
            pl.BlockSpec((1, RBLK, L), lambda i, j: (i, j, 0)),
            pl.BlockSpec((d, d), lambda i, j: (0, 0)),
            pl.BlockSpec((d, d), lambda i, j: (0, 0)),
            pl.BlockSpec((1, d), lambda i, j: (0, 0)),
        ],
        out_specs=pl.BlockSpec((1, L, d), lambda i, j: (i, 0, 0)),
        out_shape=jax.ShapeDtypeStruct((B, L, d), jnp.float32),
    )(seq, adj_matrix, W, W_self, b2d)
    return out


# speedup vs baseline: 2.3527x; 2.3527x over previous
import jax
import jax.numpy as jnp
from jax.experimental import pallas as pl


def _mp_kernel(seq_ref, adj_ref, w_ref, ws_ref, b_ref, out_ref):
    x = seq_ref[0]
    a = (adj_ref[0] != 0).astype(jnp.float32)
    h = jnp.dot(x, w_ref[...], preferred_element_type=jnp.float32)
    agg = jax.lax.dot_general(
        a, h, (((0,), (0,)), ((), ())), preferred_element_type=jnp.float32
    )
    self_term = jnp.dot(x, ws_ref[...], preferred_element_type=jnp.float32)
    out_ref[0] = agg + self_term + b_ref[...]


def kernel(seq, mask, adj_matrix, W, W_self, b):
    B, L, d = seq.shape
    del mask
    b2d = b.reshape(1, d)
    out = pl.pallas_call(
        _mp_kernel,
        grid=(B,),
        in_specs=[
            pl.BlockSpec((1, L, d), lambda i: (i, 0, 0)),
            pl.BlockSpec((1, L, L), lambda i: (i, 0, 0)),
            pl.BlockSpec((d, d), lambda i: (0, 0)),
            pl.BlockSpec((d, d), lambda i: (0, 0)),
            pl.BlockSpec((1, d), lambda i: (0, 0)),
        ],
        out_specs=pl.BlockSpec((1, L, d), lambda i: (i, 0, 0)),
        out_shape=jax.ShapeDtypeStruct((B, L, d), jnp.float32),
    )(seq, adj_matrix, W, W_self, b2d)
    return out
